# speedup vs baseline: 9.2511x; 9.2511x over previous
import functools

import jax
import jax.numpy as jnp
from jax import lax
from jax.experimental import pallas as pl
from jax.experimental.pallas import tpu as pltpu
from jax.experimental.pallas import tpu_sc as plsc

N_NODES = 10000
D = 128
G = 64

NC = 2
NS = 16
NW = NC * NS
K = 128

NPAD = 10240
ROWS_PER_TILE = NPAD // NS
ZCHUNKS = ROWS_PER_TILE // K

_mesh = plsc.VectorSubcoreMesh(core_axis_name="c", subcore_axis_name="s",
                               num_cores=NC, num_subcores=NS)


def _zero_f32_buf(ref, nrows, ncols):
    z = jnp.zeros((16,), jnp.float32)

    def body(r, _):
        for cblk in range(ncols // 16):
            ref[r, pl.ds(cblk * 16, 16)] = z
        return 0

    lax.fori_loop(0, nrows, body, 0)


def _deg_body(nchunks, dst_hbm, out_hbm, accd, didx, ones, sem):
    c = lax.axis_index("c")
    s = lax.axis_index("s")
    wid = s * NC + c

    _zero_f32_buf(ones, K, 16)
    base_r = s * ROWS_PER_TILE
    for z in range(ZCHUNKS):
        pltpu.sync_copy(ones, accd.at[pl.ds(base_r + z * K, K)])
    e0 = jnp.where(lax.iota(jnp.int32, 16) == 0, 1.0, 0.0).astype(jnp.float32)

    def fill(r, _):
        ones[r] = e0
        return 0

    lax.fori_loop(0, K, fill, 0)
    plsc.subcore_barrier()

    def body(i, _):
        base = (wid * nchunks + i) * K
        pltpu.sync_copy(dst_hbm.at[pl.ds(base, K)], didx)
        pltpu.sync_copy(ones, accd.at[didx], add=True)
        return 0

    lax.fori_loop(0, nchunks, body, 0)
    plsc.subcore_barrier()

    for z in range(ZCHUNKS):
        r0 = base_r + z * K
        pltpu.sync_copy(accd.at[pl.ds(r0, K)], ones)
        pltpu.sync_copy(ones, out_hbm.at[c, pl.ds(r0, K)])


def _prop_body(nchunks, y_hbm, src_hbm, dst_hbm, out_hbm,
               acc, sidx, didx, rows, sem):
    c = lax.axis_index("c")
    s = lax.axis_index("s")
    wid = s * NC + c

    _zero_f32_buf(rows, K, D)
    base_r = s * ROWS_PER_TILE
    for z in range(ZCHUNKS):
        pltpu.sync_copy(rows, acc.at[pl.ds(base_r + z * K, K)])
    plsc.subcore_barrier()

    def body(i, _):
        base = (wid * nchunks + i) * K
        pltpu.sync_copy(src_hbm.at[pl.ds(base, K)], sidx)
        pltpu.sync_copy(dst_hbm.at[pl.ds(base, K)], didx)
        pltpu.async_copy(y_hbm.at[sidx], rows, sem).wait()
        pltpu.sync_copy(rows, acc.at[didx], add=True)
        return 0

    lax.fori_loop(0, nchunks, body, 0)
    plsc.subcore_barrier()

    for z in range(ZCHUNKS):
        r0 = base_r + z * K
        pltpu.sync_copy(acc.at[pl.ds(r0, K)], rows)
        pltpu.sync_copy(rows, out_hbm.at[c, pl.ds(r0, K)])


def _make_deg_call(nchunks):
    return pl.kernel(
        functools.partial(_deg_body, nchunks),
        out_type=jax.ShapeDtypeStruct((NC, NPAD, 16), jnp.float32),
        mesh=_mesh,
        scratch_types=[
            pltpu.VMEM_SHARED((NPAD, 16), jnp.float32),
            pltpu.VMEM((K,), jnp.int32),
            pltpu.VMEM((K, 16), jnp.float32),
            pltpu.SemaphoreType.DMA,
        ],
    )


def _make_prop_call(nchunks):
    return pl.kernel(
        functools.partial(_prop_body, nchunks),
        out_type=jax.ShapeDtypeStruct((NC, NPAD, D), jnp.float32),
        mesh=_mesh,
        scratch_types=[
            pltpu.VMEM_SHARED((NPAD, D), jnp.float32),
            pltpu.VMEM((K,), jnp.int32),
            pltpu.VMEM((K,), jnp.int32),
            pltpu.VMEM((K, D), jnp.float32),
            pltpu.SemaphoreType.DMA,
        ],
    )


def _dinv_from(degp_ref):
    deg = degp_ref[0, :, 0:1] + degp_ref[1, :, 0:1] + 1.0
    return lax.rsqrt(deg)


def _tc_first_body(x_ref, w_ref, degp_ref, y_ref):
    dinv = _dinv_from(degp_ref)
    xw = jnp.dot(x_ref[...], w_ref[...], preferred_element_type=jnp.float32)
    y_ref[...] = xw * dinv


def _tc_mid_body(accp_ref, y_ref, degp_ref, w_ref, b_ref, out_ref):
    dinv = _dinv_from(degp_ref)
    h = dinv * (accp_ref[0] + accp_ref[1] + y_ref[...]) + b_ref[...]
    h = jnp.maximum(h, 0.0)
    out_ref[...] = jnp.dot(h, w_ref[...],
                           preferred_element_type=jnp.float32) * dinv


def _tc_final_body(accp_ref, y_ref, degp_ref, b_ref, brow_ref, bcol_ref,
                   wc_ref, bc_ref, out_ref):
    dinv = _dinv_from(degp_ref)
    h = dinv * (accp_ref[0] + accp_ref[1] + y_ref[...]) + b_ref[...]
    rid = lax.broadcasted_iota(jnp.int32, (NPAD, 1), 0)
    h = jnp.where(rid < N_NODES, h, 0.0)

    gids = lax.broadcasted_iota(jnp.int32, (G, NPAD), 0)
    oh = (brow_ref[...] == gids).astype(jnp.float32)
    sums = jnp.dot(oh, h, preferred_element_type=jnp.float32)
    counts = jnp.sum(oh, axis=1, keepdims=True)
    mean_p = sums / jnp.maximum(counts, 1.0)

    rowsel = lax.broadcasted_iota(jnp.int32, (G, 1), 0)

    def mbody(g, mp):
        m = bcol_ref[...] == g
        hm = jnp.where(m, h, -jnp.inf)
        row = jnp.max(hm, axis=0, keepdims=True)
        return jnp.where(rowsel == g, row, mp)

    max_p = lax.fori_loop(0, G, mbody, jnp.full((G, D), -jnp.inf, jnp.float32))

    cat = jnp.concatenate([mean_p, max_p], axis=1)
    out_ref[...] = jnp.dot(cat, wc_ref[...],
                           preferred_element_type=jnp.float32) + bc_ref[...]


_tc_first = pl.pallas_call(
    _tc_first_body, out_shape=jax.ShapeDtypeStruct((NPAD, D), jnp.float32))

_tc_mid = pl.pallas_call(
    _tc_mid_body, out_shape=jax.ShapeDtypeStruct((NPAD, D), jnp.float32))

_tc_final = pl.pallas_call(
    _tc_final_body, out_shape=jax.ShapeDtypeStruct((G, 128), jnp.float32))


def kernel(x, edge_index, batch, W1, b1, W2, b2, W3, b3, Wc, bc):
    n, _ = x.shape
    e = edge_index.shape[1]

    echunks = -(-e // (NW * K))
    e_pad = NW * K * echunks

    src = edge_index[0]
    dst = edge_index[1]
    src_p = jnp.pad(src, (0, e_pad - e), constant_values=N_NODES)
    dst_p = jnp.pad(dst, (0, e_pad - e), constant_values=N_NODES)

    x_pad = jnp.pad(x, ((0, NPAD - n), (0, 0)))
    brow = jnp.pad(batch, (0, NPAD - n), constant_values=G).reshape(1, NPAD)
    bcol = brow.reshape(NPAD, 1)

    deg_call = _make_deg_call(echunks)
    prop_call = _make_prop_call(echunks)

    degp = deg_call(dst_p)
    y1 = _tc_first(x_pad, W1, degp)
    acc1 = prop_call(y1, src_p, dst_p)
    y2 = _tc_mid(acc1, y1, degp, W2, b1.reshape(1, D))
    acc2 = prop_call(y2, src_p, dst_p)
    y3 = _tc_mid(acc2, y2, degp, W3, b2.reshape(1, D))
    acc3 = prop_call(y3, src_p, dst_p)

    wc_pad = jnp.pad(Wc, ((0, 0), (0, 128 - Wc.shape[1])))
    bc_pad = jnp.pad(bc, (0, 128 - bc.shape[0])).reshape(1, 128)
    out = _tc_final(acc3, y3, degp, b3.reshape(1, D), brow, bcol,
                    wc_pad, bc_pad)
    return out[:, :bc.shape[0]]


# speedup vs baseline: 11.0703x; 1.1967x over previous
import functools

import jax
import jax.numpy as jnp
from jax import lax
from jax.experimental import pallas as pl
from jax.experimental.pallas import tpu as pltpu
from jax.experimental.pallas import tpu_sc as plsc

N_NODES = 10000
D = 128
G = 64

NC = 2
NS = 16
NW = NC * NS
K = 128

NPAD = 10240
ROWS_PER_TILE = NPAD // NS
ZCHUNKS = ROWS_PER_TILE // K

_mesh = plsc.VectorSubcoreMesh(core_axis_name="c", subcore_axis_name="s",
                               num_cores=NC, num_subcores=NS)


def _zero_f32_buf(ref, nrows, ncols):
    z = jnp.zeros((16,), jnp.float32)

    def body(r, _):
        for cblk in range(ncols // 16):
            ref[r, pl.ds(cblk * 16, 16)] = z
        return 0

    lax.fori_loop(0, nrows, body, 0)


def _deg_body(nchunks, dst_hbm, out_hbm, accd, didx, ones, sem):
    c = lax.axis_index("c")
    s = lax.axis_index("s")
    wid = s * NC + c

    _zero_f32_buf(ones, K, 16)
    base_r = s * ROWS_PER_TILE
    for z in range(ZCHUNKS):
        pltpu.sync_copy(ones, accd.at[pl.ds(base_r + z * K, K)])
    e0 = jnp.where(lax.iota(jnp.int32, 16) == 0, 1.0, 0.0).astype(jnp.float32)

    def fill(r, _):
        ones[r] = e0
        return 0

    lax.fori_loop(0, K, fill, 0)
    plsc.subcore_barrier()

    def body(i, _):
        base = (wid * nchunks + i) * K
        pltpu.sync_copy(dst_hbm.at[pl.ds(base, K)], didx)
        pltpu.sync_copy(ones, accd.at[didx], add=True)
        return 0

    lax.fori_loop(0, nchunks, body, 0)
    plsc.subcore_barrier()

    for z in range(ZCHUNKS):
        r0 = base_r + z * K
        pltpu.sync_copy(accd.at[pl.ds(r0, K)], ones)
        pltpu.sync_copy(ones, out_hbm.at[c, pl.ds(r0, K)])


NSLOT = 2


def _prop_body(nchunks, y_hbm, ep_hbm, out_hbm, acc,
               pidx0, pidx1, didx0, didx1, rows0, rows1, *sems):
    c = lax.axis_index("c")
    s = lax.axis_index("s")
    wid = s * NC + c
    ngroups = nchunks // NSLOT
    tchunks = nchunks * NW

    pidx = (pidx0, pidx1)
    didx = (didx0, didx1)
    rows = (rows0, rows1)

    def copy_didx(b):
        for blk in range(K // 16):
            didx[b][pl.ds(blk * 16, 16)] = pidx[b][1, pl.ds(blk * 16, 16)]
    sem_i = sems[0:NSLOT]
    sem_g = sems[NSLOT:2 * NSLOT]
    sem_s = sems[2 * NSLOT:3 * NSLOT]

    _zero_f32_buf(rows0, K, D)
    base_r = s * ROWS_PER_TILE
    for z in range(ZCHUNKS):
        pltpu.sync_copy(rows0, acc.at[pl.ds(base_r + z * K, K)])

    plsc.subcore_barrier()

    def body(j, _):
        base = wid * nchunks + j * NSLOT
        di = [pltpu.async_copy(ep_hbm.at[base + b], pidx[b], sem_i[b])
              for b in range(NSLOT)]
        dg = []
        for b in range(NSLOT):
            di[b].wait()
            dg.append(pltpu.async_copy(y_hbm.at[pidx[b].at[0]], rows[b],
                                       sem_g[b]))
        ds_ = []
        for b in range(NSLOT):
            dg[b].wait()
            copy_didx(b)
            ds_.append(pltpu.async_copy(rows[b], acc.at[didx[b]], sem_s[b],
                                        add=True))
        for b in range(NSLOT):
            ds_[b].wait()
        return 0

    lax.fori_loop(0, ngroups, body, 0)
    plsc.subcore_barrier()

    for z in range(ZCHUNKS):
        r0 = base_r + z * K
        b = z % 2
        pltpu.sync_copy(acc.at[pl.ds(r0, K)], rows[b])
        pltpu.sync_copy(rows[b], out_hbm.at[c, pl.ds(r0, K)])


def _make_deg_call(nchunks):
    return pl.kernel(
        functools.partial(_deg_body, nchunks),
        out_type=jax.ShapeDtypeStruct((NC, NPAD, 16), jnp.float32),
        mesh=_mesh,
        scratch_types=[
            pltpu.VMEM_SHARED((NPAD, 16), jnp.float32),
            pltpu.VMEM((K,), jnp.int32),
            pltpu.VMEM((K, 16), jnp.float32),
            pltpu.SemaphoreType.DMA,
        ],
    )


def _make_prop_call(nchunks):
    return pl.kernel(
        functools.partial(_prop_body, nchunks),
        out_type=jax.ShapeDtypeStruct((NC, NPAD, D), jnp.float32),
        mesh=_mesh,
        scratch_types=(
            [pltpu.VMEM_SHARED((NPAD, D), jnp.float32)]
            + [pltpu.VMEM((2, K), jnp.int32) for _ in range(NSLOT)]
            + [pltpu.VMEM((K,), jnp.int32) for _ in range(NSLOT)]
            + [pltpu.VMEM((K, D), jnp.float32) for _ in range(NSLOT)]
            + [pltpu.SemaphoreType.DMA for _ in range(3 * NSLOT)]
        ),
    )


def _dinv_from(degp_ref):
    deg = degp_ref[0, :, 0:1] + degp_ref[1, :, 0:1] + 1.0
    return lax.rsqrt(deg)


def _tc_first_body(x_ref, w_ref, degp_ref, y_ref):
    dinv = _dinv_from(degp_ref)
    xw = jnp.dot(x_ref[...], w_ref[...], preferred_element_type=jnp.float32)
    y_ref[...] = xw * dinv


def _tc_mid_body(accp_ref, y_ref, degp_ref, w_ref, b_ref, out_ref):
    dinv = _dinv_from(degp_ref)
    h = dinv * (accp_ref[0] + accp_ref[1] + y_ref[...]) + b_ref[...]
    h = jnp.maximum(h, 0.0)
    out_ref[...] = jnp.dot(h, w_ref[...],
                           preferred_element_type=jnp.float32) * dinv


def _tc_final_body(accp_ref, y_ref, degp_ref, b_ref, brow_ref, bcol_ref,
                   wc_ref, bc_ref, out_ref):
    dinv = _dinv_from(degp_ref)
    h = dinv * (accp_ref[0] + accp_ref[1] + y_ref[...]) + b_ref[...]
    rid = lax.broadcasted_iota(jnp.int32, (NPAD, 1), 0)
    h = jnp.where(rid < N_NODES, h, 0.0)

    gids = lax.broadcasted_iota(jnp.int32, (G, NPAD), 0)
    oh = (brow_ref[...] == gids).astype(jnp.float32)
    sums = jnp.dot(oh, h, preferred_element_type=jnp.float32)
    counts = jnp.sum(oh, axis=1, keepdims=True)
    mean_p = sums / jnp.maximum(counts, 1.0)

    rowsel = lax.broadcasted_iota(jnp.int32, (G, 1), 0)

    def mbody(g, mp):
        m = bcol_ref[...] == g
        hm = jnp.where(m, h, -jnp.inf)
        row = jnp.max(hm, axis=0, keepdims=True)
        return jnp.where(rowsel == g, row, mp)

    max_p = lax.fori_loop(0, G, mbody, jnp.full((G, D), -jnp.inf, jnp.float32))

    cat = jnp.concatenate([mean_p, max_p], axis=1)
    out_ref[...] = jnp.dot(cat, wc_ref[...],
                           preferred_element_type=jnp.float32) + bc_ref[...]


_tc_first = pl.pallas_call(
    _tc_first_body, out_shape=jax.ShapeDtypeStruct((NPAD, D), jnp.float32))

_tc_mid = pl.pallas_call(
    _tc_mid_body, out_shape=jax.ShapeDtypeStruct((NPAD, D), jnp.float32))

_tc_final = pl.pallas_call(
    _tc_final_body, out_shape=jax.ShapeDtypeStruct((G, 128), jnp.float32))


def kernel(x, edge_index, batch, W1, b1, W2, b2, W3, b3, Wc, bc):
    n, _ = x.shape
    e = edge_index.shape[1]

    echunks = -(-e // (NW * K))
    e_pad = NW * K * echunks

    src = edge_index[0]
    dst = edge_index[1]
    src_p = jnp.pad(src, (0, e_pad - e), constant_values=N_NODES)
    dst_p = jnp.pad(dst, (0, e_pad - e), constant_values=N_NODES)
    tchunks = e_pad // K
    epacked = jnp.stack([src_p.reshape(tchunks, K),
                         dst_p.reshape(tchunks, K)], axis=1)

    x_pad = jnp.pad(x, ((0, NPAD - n), (0, 0)))
    brow = jnp.pad(batch, (0, NPAD - n), constant_values=G).reshape(1, NPAD)
    bcol = brow.reshape(NPAD, 1)

    deg_call = _make_deg_call(echunks)
    prop_call = _make_prop_call(echunks)

    degp = deg_call(dst_p)
    y1 = _tc_first(x_pad, W1, degp)
    acc1 = prop_call(y1, epacked)
    y2 = _tc_mid(acc1, y1, degp, W2, b1.reshape(1, D))
    acc2 = prop_call(y2, epacked)
    y3 = _tc_mid(acc2, y2, degp, W3, b2.reshape(1, D))
    acc3 = prop_call(y3, epacked)

    wc_pad = jnp.pad(Wc, ((0, 0), (0, 128 - Wc.shape[1])))
    bc_pad = jnp.pad(bc, (0, 128 - bc.shape[0])).reshape(1, 128)
    out = _tc_final(acc3, y3, degp, b3.reshape(1, D), brow, bcol,
                    wc_pad, bc_pad)
    return out[:, :bc.shape[0]]


# speedup vs baseline: 14.1029x; 1.2739x over previous
import functools

import jax
import jax.numpy as jnp
from jax import lax
from jax.experimental import pallas as pl
from jax.experimental.pallas import tpu as pltpu
from jax.experimental.pallas import tpu_sc as plsc

N_NODES = 10000
D = 128
G = 64

NC = 2
NS = 16
NW = NC * NS
K = 128

NPAD = 10240
ROWS_PER_TILE = NPAD // NS
ZCHUNKS = ROWS_PER_TILE // K

_mesh = plsc.VectorSubcoreMesh(core_axis_name="c", subcore_axis_name="s",
                               num_cores=NC, num_subcores=NS)


def _zero_f32_buf(ref, nrows, ncols):
    z = jnp.zeros((16,), jnp.float32)

    def body(r, _):
        for cblk in range(ncols // 16):
            ref[r, pl.ds(cblk * 16, 16)] = z
        return 0

    lax.fori_loop(0, nrows, body, 0)


def _deg_body(nchunks, dst_hbm, out_hbm, accd, didx, ones, sem):
    c = lax.axis_index("c")
    s = lax.axis_index("s")
    wid = s * NC + c

    _zero_f32_buf(ones, K, 16)
    base_r = s * ROWS_PER_TILE
    for z in range(ZCHUNKS):
        pltpu.sync_copy(ones, accd.at[pl.ds(base_r + z * K, K)])
    e0 = jnp.where(lax.iota(jnp.int32, 16) == 0, 1.0, 0.0).astype(jnp.float32)

    def fill(r, _):
        ones[r] = e0
        return 0

    lax.fori_loop(0, K, fill, 0)
    plsc.subcore_barrier()

    def body(i, _):
        base = (wid * nchunks + i) * K
        pltpu.sync_copy(dst_hbm.at[pl.ds(base, K)], didx)
        pltpu.sync_copy(ones, accd.at[didx], add=True)
        return 0

    lax.fori_loop(0, nchunks, body, 0)
    plsc.subcore_barrier()

    for z in range(ZCHUNKS):
        r0 = base_r + z * K
        pltpu.sync_copy(accd.at[pl.ds(r0, K)], ones)
        pltpu.sync_copy(ones, out_hbm.at[c, pl.ds(r0, K)])


NSLOT = 2
NPBUF = 4
UNROLL = 8


def _prop_body(nchunks, y_hbm, ep_hbm, out_hbm, acc,
               pidx0, pidx1, pidx2, pidx3, didx0, didx1, didx2, didx3,
               rows0, rows1, *sems):
    c = lax.axis_index("c")
    s = lax.axis_index("s")
    wid = s * NC + c
    ngroups = nchunks // UNROLL

    pidx = (pidx0, pidx1, pidx2, pidx3)
    didx = (didx0, didx1, didx2, didx3)
    rows = (rows0, rows1)

    def copy_didx(p):
        for blk in range(K // 16):
            didx[p][pl.ds(blk * 16, 16)] = pidx[p][1, pl.ds(blk * 16, 16)]

    sem_i = sems[0:NPBUF]
    sem_g = sems[NPBUF:NPBUF + NSLOT]
    sem_s = sems[NPBUF + NSLOT:NPBUF + 2 * NSLOT]

    _zero_f32_buf(rows0, K, D)
    base_r = s * ROWS_PER_TILE
    for z in range(ZCHUNKS):
        pltpu.sync_copy(rows0, acc.at[pl.ds(base_r + z * K, K)])

    plsc.subcore_barrier()

    def body(j, _):
        base = wid * nchunks + j * UNROLL
        di = [pltpu.async_copy(ep_hbm.at[base + p], pidx[p], sem_i[p])
              for p in range(NPBUF)]
        dg = [None] * UNROLL
        ds_ = [None] * UNROLL
        for step in range(UNROLL + 1):
            if step < UNROLL:
                b, p = step % NSLOT, step % NPBUF
                if step >= NSLOT:
                    ds_[step - NSLOT].wait()
                di[p].wait()
                dg[step] = pltpu.async_copy(y_hbm.at[pidx[p].at[0]],
                                            rows[b], sem_g[b])
            if step >= 1:
                cc = step - 1
                b, p = cc % NSLOT, cc % NPBUF
                dg[cc].wait()
                copy_didx(p)
                ds_[cc] = pltpu.async_copy(rows[b], acc.at[didx[p]],
                                           sem_s[b], add=True)
                if cc + NPBUF < UNROLL:
                    di[p] = pltpu.async_copy(ep_hbm.at[base + cc + NPBUF],
                                             pidx[p], sem_i[p])
        ds_[UNROLL - 2].wait()
        ds_[UNROLL - 1].wait()
        return 0

    lax.fori_loop(0, ngroups, body, 0)
    plsc.subcore_barrier()

    for z in range(ZCHUNKS):
        r0 = base_r + z * K
        b = z % 2
        pltpu.sync_copy(acc.at[pl.ds(r0, K)], rows[b])
        pltpu.sync_copy(rows[b], out_hbm.at[c, pl.ds(r0, K)])


def _make_deg_call(nchunks):
    return pl.kernel(
        functools.partial(_deg_body, nchunks),
        out_type=jax.ShapeDtypeStruct((NC, NPAD, 16), jnp.float32),
        mesh=_mesh,
        scratch_types=[
            pltpu.VMEM_SHARED((NPAD, 16), jnp.float32),
            pltpu.VMEM((K,), jnp.int32),
            pltpu.VMEM((K, 16), jnp.float32),
            pltpu.SemaphoreType.DMA,
        ],
    )


def _make_prop_call(nchunks):
    return pl.kernel(
        functools.partial(_prop_body, nchunks),
        out_type=jax.ShapeDtypeStruct((NC, NPAD, D), jnp.float32),
        mesh=_mesh,
        scratch_types=(
            [pltpu.VMEM_SHARED((NPAD, D), jnp.float32)]
            + [pltpu.VMEM((2, K), jnp.int32) for _ in range(NPBUF)]
            + [pltpu.VMEM((K,), jnp.int32) for _ in range(NPBUF)]
            + [pltpu.VMEM((K, D), jnp.float32) for _ in range(NSLOT)]
            + [pltpu.SemaphoreType.DMA for _ in range(NPBUF + 2 * NSLOT)]
        ),
    )


def _dinv_from(degp_ref):
    deg = degp_ref[0, :, 0:1] + degp_ref[1, :, 0:1] + 1.0
    return lax.rsqrt(deg)


def _tc_first_body(x_ref, w_ref, degp_ref, y_ref):
    dinv = _dinv_from(degp_ref)
    xw = jnp.dot(x_ref[...], w_ref[...], preferred_element_type=jnp.float32)
    y_ref[...] = xw * dinv


def _tc_mid_body(accp_ref, y_ref, degp_ref, w_ref, b_ref, out_ref):
    dinv = _dinv_from(degp_ref)
    h = dinv * (accp_ref[0] + accp_ref[1] + y_ref[...]) + b_ref[...]
    h = jnp.maximum(h, 0.0)
    out_ref[...] = jnp.dot(h, w_ref[...],
                           preferred_element_type=jnp.float32) * dinv


def _tc_final_body(accp_ref, y_ref, degp_ref, b_ref, brow_ref, bcol_ref,
                   wc_ref, bc_ref, out_ref):
    dinv = _dinv_from(degp_ref)
    h = dinv * (accp_ref[0] + accp_ref[1] + y_ref[...]) + b_ref[...]
    rid = lax.broadcasted_iota(jnp.int32, (NPAD, 1), 0)
    h = jnp.where(rid < N_NODES, h, 0.0)

    gids = lax.broadcasted_iota(jnp.int32, (G, NPAD), 0)
    oh = (brow_ref[...] == gids).astype(jnp.float32)
    sums = jnp.dot(oh, h, preferred_element_type=jnp.float32)
    counts = jnp.sum(oh, axis=1, keepdims=True)
    mean_p = sums / jnp.maximum(counts, 1.0)

    rowsel = lax.broadcasted_iota(jnp.int32, (G, 1), 0)

    def mbody(g, mp):
        m = bcol_ref[...] == g
        hm = jnp.where(m, h, -jnp.inf)
        row = jnp.max(hm, axis=0, keepdims=True)
        return jnp.where(rowsel == g, row, mp)

    max_p = lax.fori_loop(0, G, mbody, jnp.full((G, D), -jnp.inf, jnp.float32))

    cat = jnp.concatenate([mean_p, max_p], axis=1)
    out_ref[...] = jnp.dot(cat, wc_ref[...],
                           preferred_element_type=jnp.float32) + bc_ref[...]


_tc_first = pl.pallas_call(
    _tc_first_body, out_shape=jax.ShapeDtypeStruct((NPAD, D), jnp.float32))

_tc_mid = pl.pallas_call(
    _tc_mid_body, out_shape=jax.ShapeDtypeStruct((NPAD, D), jnp.float32))

_tc_final = pl.pallas_call(
    _tc_final_body, out_shape=jax.ShapeDtypeStruct((G, 128), jnp.float32))


def kernel(x, edge_index, batch, W1, b1, W2, b2, W3, b3, Wc, bc):
    n, _ = x.shape
    e = edge_index.shape[1]

    echunks = -(-e // (NW * K))
    e_pad = NW * K * echunks

    src = edge_index[0]
    dst = edge_index[1]
    src_p = jnp.pad(src, (0, e_pad - e), constant_values=N_NODES)
    dst_p = jnp.pad(dst, (0, e_pad - e), constant_values=N_NODES)
    tchunks = e_pad // K
    epacked = jnp.stack([src_p.reshape(tchunks, K),
                         dst_p.reshape(tchunks, K)], axis=1)

    x_pad = jnp.pad(x, ((0, NPAD - n), (0, 0)))
    brow = jnp.pad(batch, (0, NPAD - n), constant_values=G).reshape(1, NPAD)
    bcol = brow.reshape(NPAD, 1)

    deg_call = _make_deg_call(echunks)
    prop_call = _make_prop_call(echunks)

    degp = deg_call(dst_p)
    y1 = _tc_first(x_pad, W1, degp)
    acc1 = prop_call(y1, epacked)
    y2 = _tc_mid(acc1, y1, degp, W2, b1.reshape(1, D))
    acc2 = prop_call(y2, epacked)
    y3 = _tc_mid(acc2, y2, degp, W3, b2.reshape(1, D))
    acc3 = prop_call(y3, epacked)

    wc_pad = jnp.pad(Wc, ((0, 0), (0, 128 - Wc.shape[1])))
    bc_pad = jnp.pad(bc, (0, 128 - bc.shape[0])).reshape(1, 128)
    out = _tc_final(acc3, y3, degp, b3.reshape(1, D), brow, bcol,
                    wc_pad, bc_pad)
    return out[:, :bc.shape[0]]


# speedup vs baseline: 22.6585x; 1.6067x over previous
import functools

import jax
import jax.numpy as jnp
from jax import lax
from jax.experimental import pallas as pl
from jax.experimental.pallas import tpu as pltpu
from jax.experimental.pallas import tpu_sc as plsc

N_NODES = 10000
D = 128
G = 64

NC = 2
NS = 16
NW = NC * NS
K = 128

NPAD = 10240
ROWS_PER_TILE = NPAD // NS
ZCHUNKS = ROWS_PER_TILE // K

_mesh = plsc.VectorSubcoreMesh(core_axis_name="c", subcore_axis_name="s",
                               num_cores=NC, num_subcores=NS)


def _zero_f32_buf(ref, nrows, ncols):
    z = jnp.zeros((16,), jnp.float32)

    def body(r, _):
        for cblk in range(ncols // 16):
            ref[r, pl.ds(cblk * 16, 16)] = z
        return 0

    lax.fori_loop(0, nrows, body, 0)


def _deg_body(nchunks, dst_hbm, out_hbm, accd, didx, ones, sem):
    c = lax.axis_index("c")
    s = lax.axis_index("s")
    wid = s * NC + c

    _zero_f32_buf(ones, K, 16)
    base_r = s * ROWS_PER_TILE
    for z in range(ZCHUNKS):
        pltpu.sync_copy(ones, accd.at[pl.ds(base_r + z * K, K)])
    e0 = jnp.where(lax.iota(jnp.int32, 16) == 0, 1.0, 0.0).astype(jnp.float32)

    def fill(r, _):
        ones[r] = e0
        return 0

    lax.fori_loop(0, K, fill, 0)
    plsc.subcore_barrier()

    def body(i, _):
        base = (wid * nchunks + i) * K
        pltpu.sync_copy(dst_hbm.at[pl.ds(base, K)], didx)
        pltpu.sync_copy(ones, accd.at[didx], add=True)
        return 0

    lax.fori_loop(0, nchunks, body, 0)
    plsc.subcore_barrier()

    for z in range(ZCHUNKS):
        r0 = base_r + z * K
        pltpu.sync_copy(accd.at[pl.ds(r0, K)], ones)
        pltpu.sync_copy(ones, out_hbm.at[c, pl.ds(r0, K)])


NSLOT = 2
NPBUF = 4
UNROLL = 8


def _prop_body(nchunks, y_hbm, ep_hbm, out_hbm, acc,
               pidx0, pidx1, pidx2, pidx3, didx0, didx1, didx2, didx3,
               rows0, rows1, *sems):
    c = lax.axis_index("c")
    s = lax.axis_index("s")
    wid = s * NC + c
    ngroups = nchunks // UNROLL

    pidx = (pidx0, pidx1, pidx2, pidx3)
    didx = (didx0, didx1, didx2, didx3)
    rows = (rows0, rows1)

    def copy_didx(p):
        for blk in range(K // 16):
            didx[p][pl.ds(blk * 16, 16)] = pidx[p][1, pl.ds(blk * 16, 16)]

    sem_i = sems[0:NPBUF]
    sem_g = sems[NPBUF:NPBUF + NSLOT]
    sem_s = sems[NPBUF + NSLOT:NPBUF + 2 * NSLOT]

    _zero_f32_buf(rows0, K, D)
    base_r = s * ROWS_PER_TILE
    for z in range(ZCHUNKS):
        pltpu.sync_copy(rows0, acc.at[pl.ds(base_r + z * K, K)])

    plsc.subcore_barrier()

    def body(j, _):
        base = wid * nchunks + j * UNROLL
        di = [pltpu.async_copy(ep_hbm.at[base + p], pidx[p], sem_i[p])
              for p in range(NPBUF)]
        dg = [None] * UNROLL
        ds_ = [None] * UNROLL
        for step in range(UNROLL + 1):
            if step < UNROLL:
                b, p = step % NSLOT, step % NPBUF
                if step >= NSLOT:
                    ds_[step - NSLOT].wait()
                di[p].wait()
                dg[step] = pltpu.async_copy(y_hbm.at[pidx[p].at[0]],
                                            rows[b], sem_g[b])
            if step >= 1:
                cc = step - 1
                b, p = cc % NSLOT, cc % NPBUF
                dg[cc].wait()
                copy_didx(p)
                ds_[cc] = pltpu.async_copy(rows[b], acc.at[didx[p]],
                                           sem_s[b], add=True)
                if cc + NPBUF < UNROLL:
                    di[p] = pltpu.async_copy(ep_hbm.at[base + cc + NPBUF],
                                             pidx[p], sem_i[p])
        ds_[UNROLL - 2].wait()
        ds_[UNROLL - 1].wait()
        return 0

    lax.fori_loop(0, ngroups, body, 0)
    plsc.subcore_barrier()

    for z in range(ZCHUNKS):
        r0 = base_r + z * K
        b = z % 2
        pltpu.sync_copy(acc.at[pl.ds(r0, K)], rows[b])
        pltpu.sync_copy(rows[b], out_hbm.at[c, pl.ds(r0, K)])


def _make_deg_call(nchunks):
    return pl.kernel(
        functools.partial(_deg_body, nchunks),
        out_type=jax.ShapeDtypeStruct((NC, NPAD, 16), jnp.float32),
        mesh=_mesh,
        scratch_types=[
            pltpu.VMEM_SHARED((NPAD, 16), jnp.float32),
            pltpu.VMEM((K,), jnp.int32),
            pltpu.VMEM((K, 16), jnp.float32),
            pltpu.SemaphoreType.DMA,
        ],
    )


def _make_prop_call(nchunks):
    return pl.kernel(
        functools.partial(_prop_body, nchunks),
        out_type=jax.ShapeDtypeStruct((NC, NPAD, D), jnp.float32),
        mesh=_mesh,
        scratch_types=(
            [pltpu.VMEM_SHARED((NPAD, D), jnp.float32)]
            + [pltpu.VMEM((2, K), jnp.int32) for _ in range(NPBUF)]
            + [pltpu.VMEM((K,), jnp.int32) for _ in range(NPBUF)]
            + [pltpu.VMEM((K, D), jnp.float32) for _ in range(NSLOT)]
            + [pltpu.SemaphoreType.DMA for _ in range(NPBUF + 2 * NSLOT)]
        ),
    )


def _dinv_from(degp_ref):
    deg = degp_ref[0, :, 0:1] + degp_ref[1, :, 0:1] + 1.0
    return lax.rsqrt(deg)


def _tc_first_body(x_ref, w_ref, degp_ref, y_ref):
    dinv = _dinv_from(degp_ref)
    xw = jnp.dot(x_ref[...], w_ref[...], preferred_element_type=jnp.float32)
    y_ref[...] = xw * dinv


def _tc_mid_body(accp_ref, y_ref, degp_ref, w_ref, b_ref, out_ref):
    dinv = _dinv_from(degp_ref)
    h = dinv * (accp_ref[0] + accp_ref[1] + y_ref[...]) + b_ref[...]
    h = jnp.maximum(h, 0.0)
    out_ref[...] = jnp.dot(h, w_ref[...],
                           preferred_element_type=jnp.float32) * dinv


def _tc_final_body(accp_ref, y_ref, degp_ref, b_ref, brow_ref, bcol_ref,
                   wc_ref, bc_ref, out_ref):
    dinv = _dinv_from(degp_ref)
    h = dinv * (accp_ref[0] + accp_ref[1] + y_ref[...]) + b_ref[...]
    rid = lax.broadcasted_iota(jnp.int32, (NPAD, 1), 0)
    h = jnp.where(rid < N_NODES, h, 0.0)

    gids = lax.broadcasted_iota(jnp.int32, (G, NPAD), 0)
    oh = (brow_ref[...] == gids).astype(jnp.float32)
    sums = jnp.dot(oh, h, preferred_element_type=jnp.float32)
    counts = jnp.sum(oh, axis=1, keepdims=True)
    mean_p = sums / jnp.maximum(counts, 1.0)

    rowsel = lax.broadcasted_iota(jnp.int32, (G, 1), 0)

    def mbody(g, mp):
        m = bcol_ref[...] == g
        hm = jnp.where(m, h, -jnp.inf)
        row = jnp.max(hm, axis=0, keepdims=True)
        return jnp.where(rowsel == g, row, mp)

    max_p = lax.fori_loop(0, G, mbody, jnp.full((G, D), -jnp.inf, jnp.float32))

    cat = jnp.concatenate([mean_p, max_p], axis=1)
    out_ref[...] = jnp.dot(cat, wc_ref[...],
                           preferred_element_type=jnp.float32) + bc_ref[...]


_tc_first = pl.pallas_call(
    _tc_first_body, out_shape=jax.ShapeDtypeStruct((NPAD, D), jnp.float32))

_tc_mid = pl.pallas_call(
    _tc_mid_body, out_shape=jax.ShapeDtypeStruct((NPAD, D), jnp.float32))

_tc_final = pl.pallas_call(
    _tc_final_body, out_shape=jax.ShapeDtypeStruct((G, 128), jnp.float32))


def kernel(x, edge_index, batch, W1, b1, W2, b2, W3, b3, Wc, bc):
    n, _ = x.shape
    e = edge_index.shape[1]

    echunks = -(-e // (NW * K))
    e_pad = NW * K * echunks

    src = edge_index[0]
    dst = edge_index[1]
    trash = N_NODES + jnp.arange(e_pad - e, dtype=jnp.int32) % (NPAD - N_NODES)
    src_p = jnp.concatenate([src, trash])
    dst_p = jnp.concatenate([dst, trash])
    tchunks = e_pad // K
    epacked = jnp.stack([src_p.reshape(tchunks, K),
                         dst_p.reshape(tchunks, K)], axis=1)

    x_pad = jnp.pad(x, ((0, NPAD - n), (0, 0)))
    brow = jnp.pad(batch, (0, NPAD - n), constant_values=G).reshape(1, NPAD)
    bcol = brow.reshape(NPAD, 1)

    deg_call = _make_deg_call(echunks)
    prop_call = _make_prop_call(echunks)

    degp = deg_call(dst_p)
    y1 = _tc_first(x_pad, W1, degp)
    acc1 = prop_call(y1, epacked)
    y2 = _tc_mid(acc1, y1, degp, W2, b1.reshape(1, D))
    acc2 = prop_call(y2, epacked)
    y3 = _tc_mid(acc2, y2, degp, W3, b2.reshape(1, D))
    acc3 = prop_call(y3, epacked)

    wc_pad = jnp.pad(Wc, ((0, 0), (0, 128 - Wc.shape[1])))
    bc_pad = jnp.pad(bc, (0, 128 - bc.shape[0])).reshape(1, 128)
    out = _tc_final(acc3, y3, degp, b3.reshape(1, D), brow, bcol,
                    wc_pad, bc_pad)
    return out[:, :bc.shape[0]]
